# baseline (speedup 1.0000x reference)
import functools

import jax
import jax.numpy as jnp
from jax import lax
from jax.experimental import pallas as pl
from jax.experimental.pallas import tpu as pltpu
from jax.experimental.pallas import tpu_sc as plsc

D = 64
B = 16384
NC, NS = 2, 16
NW = NC * NS
NTC = 7813
PER_W = 245
W = 5
WL = W * 128
NWIN = 49
ICH = 2048
SLOTS = 64
OUT_PAD = B + 8
IDCAP = B + 16


@functools.partial(
    pl.kernel,
    mesh=plsc.VectorSubcoreMesh(core_axis_name="c", subcore_axis_name="s"),
    out_type=jax.ShapeDtypeStruct((OUT_PAD, 128), jnp.float32),
    scratch_types=[
        pltpu.VMEM((ICH,), jnp.int32),
        pltpu.VMEM((IDCAP,), jnp.int32),
        pltpu.VMEM((IDCAP,), jnp.int32),
        pltpu.VMEM((SLOTS, 128), jnp.float32),
        pltpu.VMEM((SLOTS,), jnp.int32),
        pltpu.VMEM((2, D, WL), jnp.float32),
        [pltpu.SemaphoreType.DMA] * 2,
        pltpu.SemaphoreType.DMA,
    ],
    compiler_params=pltpu.CompilerParams(
        use_tc_tiling_on_sc=True, needs_layout_passes=False
    ),
)
def _sc_gather_stream(tableT, idx_hbm, out_hbm, idx_buf, packed, match_buf,
                      stage, pos_stage, ring, sems, ssem):
    wid = lax.axis_index("s") * NC + lax.axis_index("c")
    lo = wid * PER_W
    mycnt = jnp.minimum(jnp.int32(PER_W), jnp.int32(NTC) - lo)
    hi = lo + mycnt
    lane_iota = lax.broadcasted_iota(jnp.int32, (16,), 0)

    def chunk(c, cnt):
        pltpu.sync_copy(idx_hbm.at[pl.ds(pl.multiple_of(c * ICH, ICH), ICH)],
                        idx_buf)
        def inner(kk, cnt):
            vec = idx_buf[pl.ds(pl.multiple_of(kk * 16, 16), 16)]
            tc = vec >> 7
            m = (tc >= lo) & (tc < hi)
            posv = c * ICH + kk * 16 + lane_iota
            pk = ((tc - lo) << 21) | ((vec & 127) << 14) | posv
            plsc.store_compressed(packed.at[pl.ds(cnt, 16)], pk, mask=m)
            npop = plsc.all_reduce_population_count(m)
            return cnt + lax.reduce_max(npop, axes=(0,))
        return lax.fori_loop(0, ICH // 16, inner, cnt)

    cnt = lax.fori_loop(0, B // ICH, chunk, jnp.int32(0))
    nck = (cnt + 15) >> 4

    def win_start(w):
        return jnp.minimum(lo + w * W, jnp.int32(NTC - W))

    def fire(w, p):
        col = pl.multiple_of(win_start(w) * 128, 128)
        pltpu.async_copy(tableT.at[:, pl.ds(col, WL)], ring.at[p], sems[p])

    def flush(s):
        for k in range(SLOTS // 16):
            slotv = k * 16 + lane_iota
            plsc.store_scatter(pos_stage, [slotv],
                               jnp.full((16,), B, jnp.int32),
                               mask=slotv >= s)
        pltpu.async_copy(stage, out_hbm.at[pos_stage], ssem).wait()
        return jnp.int32(0)

    def process_window(w, p, s):
        pltpu.make_async_copy(tableT.at[:, pl.ds(0, WL)],
                              ring.at[p], sems[p]).wait()
        col0 = w * W
        fetched0 = win_start(w) - lo
        pv = jnp.full((16,), 0, jnp.int32) + p

        def scan_chunk(ki, nm):
            valid = (ki * 16 + lane_iota) < cnt
            pkv = packed[pl.ds(ki * 16, 16)]
            pkt = pkv >> 21
            m = (pkt >= col0) & (pkt < col0 + W) & valid
            plsc.store_compressed(match_buf.at[pl.ds(nm, 16)], pkv, mask=m)
            npop = plsc.all_reduce_population_count(m)
            return nm + lax.reduce_max(npop, axes=(0,))

        nm = lax.fori_loop(0, nck, scan_chunk, jnp.int32(0))
        nb = (nm + 15) >> 4

        def ext_batch(bi, s):
            s = lax.cond(s > SLOTS - 16, flush, lambda x: x, s)
            rem = nm - bi * 16
            mv = lane_iota < rem
            pkb = match_buf[pl.ds(bi * 16, 16)]
            off = ((pkb >> 21) - fetched0) * 128 + ((pkb >> 14) & 127)
            posb = pkb & 16383
            sv = s + lane_iota
            for c in range(D):
                cv = jnp.full((16,), 0, jnp.int32) + c
                vals = plsc.load_gather(ring, [pv, cv, off], mask=mv)
                plsc.store_scatter(stage, [sv, cv], vals, mask=mv)
            plsc.store_scatter(pos_stage, [sv], posb, mask=mv)
            return s + jnp.minimum(rem, 16)

        return lax.fori_loop(0, nb, ext_batch, s)

    fire(jnp.int32(0), 0)
    fire(jnp.int32(1), 1)

    def outer(q, s):
        w0 = 2 * q
        s = process_window(w0, 0, s)
        @pl.when(w0 + 2 < NWIN)
        def _():
            fire(w0 + 2, 0)
        s = process_window(w0 + 1, 1, s)
        @pl.when(w0 + 3 < NWIN)
        def _():
            fire(w0 + 3, 1)
        return s

    s = lax.fori_loop(0, (NWIN - 1) // 2, outer, jnp.int32(0))
    s = process_window(jnp.int32(NWIN - 1), 0, s)
    flush(s)


def kernel(data, indices):
    idx = indices.astype(jnp.int32)
    padded = _sc_gather_stream(data.T, idx)
    return padded[:B, :D]


# speedup vs baseline: 1.9417x; 1.9417x over previous
import functools

import jax
import jax.numpy as jnp
from jax import lax
from jax.experimental import pallas as pl
from jax.experimental.pallas import tpu as pltpu
from jax.experimental.pallas import tpu_sc as plsc

D = 64
B = 16384
NC, NS = 2, 16
NW = NC * NS
NTC = 7813
PER_W = 245
W = 6
WL = W * 128
NWIN = 41
ICH = 2048
SLOTS = 64
OUT_PAD = B + 8
IDCAP = B + 16


@functools.partial(
    pl.kernel,
    mesh=plsc.VectorSubcoreMesh(core_axis_name="c", subcore_axis_name="s"),
    out_type=jax.ShapeDtypeStruct((OUT_PAD, 128), jnp.float32),
    scratch_types=[
        pltpu.VMEM((ICH,), jnp.int32),
        pltpu.VMEM((IDCAP,), jnp.int32),
        pltpu.VMEM((SLOTS, 128), jnp.float32),
        pltpu.VMEM((SLOTS,), jnp.int32),
        pltpu.VMEM((2, D, WL), jnp.float32),
        [pltpu.SemaphoreType.DMA] * 2,
        pltpu.SemaphoreType.DMA,
    ],
    compiler_params=pltpu.CompilerParams(
        use_tc_tiling_on_sc=True, needs_layout_passes=False
    ),
)
def _sc_gather_stream(tableT, idx_hbm, out_hbm,
                      idx_buf, packed, stage, pos_stage, ring, sems, ssem):
    wid = lax.axis_index("s") * NC + lax.axis_index("c")
    lo = wid * PER_W
    mycnt = jnp.minimum(jnp.int32(PER_W), jnp.int32(NTC) - lo)
    hi = lo + mycnt
    lane_iota = lax.broadcasted_iota(jnp.int32, (16,), 0)

    def chunk(c, cnt):
        pltpu.sync_copy(idx_hbm.at[pl.ds(pl.multiple_of(c * ICH, ICH), ICH)],
                        idx_buf)
        def inner(kk, cnt):
            vec = idx_buf[pl.ds(pl.multiple_of(kk * 16, 16), 16)]
            tc = vec >> 7
            m = (tc >= lo) & (tc < hi)
            posv = c * ICH + kk * 16 + lane_iota
            pk = ((tc - lo) << 21) | ((vec & 127) << 14) | posv
            plsc.store_compressed(packed.at[pl.ds(cnt, 16)], pk, mask=m)
            npop = plsc.all_reduce_population_count(m)
            return cnt + lax.reduce_max(npop, axes=(0,))
        return lax.fori_loop(0, ICH // 16, inner, cnt)

    cnt = lax.fori_loop(0, B // ICH, chunk, jnp.int32(0))
    nck = (cnt + 15) >> 4

    def win_start(w):
        return jnp.minimum(lo + w * W, jnp.int32(NTC - W))

    def fire(w, p):
        col = pl.multiple_of(win_start(w) * 128, 128)
        pltpu.async_copy(tableT.at[:, pl.ds(col, WL)], ring.at[p], sems[p])

    def flush(s):
        for k in range(SLOTS // 16):
            slotv = k * 16 + lane_iota
            plsc.store_scatter(pos_stage, [slotv],
                               jnp.full((16,), B, jnp.int32),
                               mask=slotv >= s)
        pltpu.async_copy(stage, out_hbm.at[pos_stage], ssem).wait()
        return jnp.int32(0)

    def process_window(w, p, s):
        pltpu.make_async_copy(tableT.at[:, pl.ds(0, WL)],
                              ring.at[p], sems[p]).wait()
        col0 = w * W
        fetched0 = win_start(w) - lo
        pv = jnp.full((16,), 0, jnp.int32) + p

        def scan_chunk(ki, s):
            valid = (ki * 16 + lane_iota) < cnt
            pkv = packed[pl.ds(ki * 16, 16)]
            pkt = pkv >> 21
            m = (pkt >= col0) & (pkt < col0 + W) & valid
            npop = lax.reduce_max(
                plsc.all_reduce_population_count(m), axes=(0,))

            def ext(i, st):
                m, s = st
                j = lax.reduce_max(plsc.all_reduce_ffs(m), axes=(0,))
                sel = lane_iota == j
                pk = lax.reduce_max(
                    jnp.where(sel, pkv, jnp.int32(-2147483648)), axes=(0,))
                off = ((pk >> 21) - fetched0) * 128 + ((pk >> 14) & 127)
                pos = pk & 16383
                ov = jnp.full((16,), 0, jnp.int32) + off
                sv = jnp.full((16,), 0, jnp.int32) + s
                for c4 in range(D // 16):
                    row_idx = c4 * 16 + lane_iota
                    vals = plsc.load_gather(ring, [pv, row_idx, ov])
                    plsc.store_scatter(stage, [sv, row_idx], vals)
                plsc.store_scatter(pos_stage, [sv],
                                   jnp.full((16,), 0, jnp.int32) + pos,
                                   mask=lane_iota == 0)
                s = s + 1
                s = lax.cond(s >= SLOTS, flush, lambda x: x, s)
                return m & (~sel), s

            _, s = lax.fori_loop(0, npop, ext, (m, s))
            return s

        return lax.fori_loop(0, nck, scan_chunk, s)

    fire(jnp.int32(0), 0)
    fire(jnp.int32(1), 1)

    def outer(q, s):
        w0 = 2 * q
        s = process_window(w0, 0, s)
        @pl.when(w0 + 2 < NWIN)
        def _():
            fire(w0 + 2, 0)
        s = process_window(w0 + 1, 1, s)
        @pl.when(w0 + 3 < NWIN)
        def _():
            fire(w0 + 3, 1)
        return s

    s = lax.fori_loop(0, (NWIN - 1) // 2, outer, jnp.int32(0))
    s = process_window(jnp.int32(NWIN - 1), 0, s)
    flush(s)


def kernel(data, indices):
    idx = indices.astype(jnp.int32)
    padded = _sc_gather_stream(data.T, idx)
    return padded[:B, :D]


# speedup vs baseline: 1.9977x; 1.0288x over previous
import functools

import jax
import jax.numpy as jnp
from jax import lax
from jax.experimental import pallas as pl
from jax.experimental.pallas import tpu as pltpu
from jax.experimental.pallas import tpu_sc as plsc

D = 64
B = 16384
NC, NS = 2, 16
NW = NC * NS
NTC = 7813
PER_W = 245
W = 6
WL = W * 128
NWIN = 41
ICH = 2048
CU = 8
SU = 4
SLOTS = 64
OUT_PAD = B + 8
IDCAP = B + 80


@functools.partial(
    pl.kernel,
    mesh=plsc.VectorSubcoreMesh(core_axis_name="c", subcore_axis_name="s"),
    out_type=jax.ShapeDtypeStruct((OUT_PAD, 128), jnp.float32),
    scratch_types=[
        pltpu.VMEM((ICH,), jnp.int32),
        pltpu.VMEM((IDCAP,), jnp.int32),
        pltpu.VMEM((SLOTS, 128), jnp.float32),
        pltpu.VMEM((SLOTS,), jnp.int32),
        pltpu.VMEM((2, D, WL), jnp.float32),
        [pltpu.SemaphoreType.DMA] * 2,
        pltpu.SemaphoreType.DMA,
    ],
    compiler_params=pltpu.CompilerParams(
        use_tc_tiling_on_sc=True, needs_layout_passes=False
    ),
)
def _sc_gather_stream(tableT, idx_hbm, out_hbm,
                      idx_buf, packed, stage, pos_stage, ring, sems, ssem):
    wid = lax.axis_index("s") * NC + lax.axis_index("c")
    lo = wid * PER_W
    mycnt = jnp.minimum(jnp.int32(PER_W), jnp.int32(NTC) - lo)
    hi = lo + mycnt
    lane_iota = lax.broadcasted_iota(jnp.int32, (16,), 0)

    def chunk(c, cnt):
        pltpu.sync_copy(idx_hbm.at[pl.ds(pl.multiple_of(c * ICH, ICH), ICH)],
                        idx_buf)
        def inner(kk, cnt):
            pks, ms, nps = [], [], []
            for u in range(CU):
                vec = idx_buf[pl.ds(
                    pl.multiple_of((kk * CU + u) * 16, 16), 16)]
                tc = vec >> 7
                m = (tc >= lo) & (tc < hi)
                posv = c * ICH + (kk * CU + u) * 16 + lane_iota
                pks.append(((tc - lo) << 21) | ((vec & 127) << 14) | posv)
                ms.append(m)
                nps.append(plsc.all_reduce_population_count(m))
            ns = [lax.reduce_max(np_, axes=(0,)) for np_ in nps]
            off = cnt
            for u in range(CU):
                plsc.store_compressed(packed.at[pl.ds(off, 16)],
                                      pks[u], mask=ms[u])
                off = off + ns[u]
            return off
        return lax.fori_loop(0, ICH // (16 * CU), inner, cnt)

    cnt = lax.fori_loop(0, B // ICH, chunk, jnp.int32(0))
    nck = (cnt + 16 * SU - 1) // (16 * SU)

    def win_start(w):
        return jnp.minimum(lo + w * W, jnp.int32(NTC - W))

    def fire(w, p):
        col = pl.multiple_of(win_start(w) * 128, 128)
        pltpu.async_copy(tableT.at[:, pl.ds(col, WL)], ring.at[p], sems[p])

    def flush(s):
        for k in range(SLOTS // 16):
            slotv = k * 16 + lane_iota
            plsc.store_scatter(pos_stage, [slotv],
                               jnp.full((16,), B, jnp.int32),
                               mask=slotv >= s)
        pltpu.async_copy(stage, out_hbm.at[pos_stage], ssem).wait()
        return jnp.int32(0)

    def process_window(w, p, s):
        pltpu.make_async_copy(tableT.at[:, pl.ds(0, WL)],
                              ring.at[p], sems[p]).wait()
        col0 = w * W
        fetched0 = win_start(w) - lo
        pv = jnp.full((16,), 0, jnp.int32) + p

        def ext_all(pkv, m, npop, s):
            def ext(i, st):
                m, s = st
                j = lax.reduce_max(plsc.all_reduce_ffs(m), axes=(0,))
                sel = lane_iota == j
                pk = lax.reduce_max(
                    jnp.where(sel, pkv, jnp.int32(-2147483648)), axes=(0,))
                off = ((pk >> 21) - fetched0) * 128 + ((pk >> 14) & 127)
                pos = pk & 16383
                ov = jnp.full((16,), 0, jnp.int32) + off
                sv = jnp.full((16,), 0, jnp.int32) + s
                for c4 in range(D // 16):
                    row_idx = c4 * 16 + lane_iota
                    vals = plsc.load_gather(ring, [pv, row_idx, ov])
                    plsc.store_scatter(stage, [sv, row_idx], vals)
                plsc.store_scatter(pos_stage, [sv],
                                   jnp.full((16,), 0, jnp.int32) + pos,
                                   mask=lane_iota == 0)
                s = s + 1
                s = lax.cond(s >= SLOTS, flush, lambda x: x, s)
                return m & (~sel), s
            _, s = lax.fori_loop(0, npop, ext, (m, s))
            return s

        def scan_chunk(ki, s):
            pkvs, msk, nps = [], [], []
            for u in range(SU):
                base = (ki * SU + u) * 16
                valid = (base + lane_iota) < cnt
                pkv = packed[pl.ds(base, 16)]
                pkt = pkv >> 21
                m = (pkt >= col0) & (pkt < col0 + W) & valid
                pkvs.append(pkv)
                msk.append(m)
                nps.append(plsc.all_reduce_population_count(m))
            ns = [lax.reduce_max(np_, axes=(0,)) for np_ in nps]
            for u in range(SU):
                s = ext_all(pkvs[u], msk[u], ns[u], s)
            return s

        return lax.fori_loop(0, nck, scan_chunk, s)

    fire(jnp.int32(0), 0)
    fire(jnp.int32(1), 1)

    def outer(q, s):
        w0 = 2 * q
        s = process_window(w0, 0, s)
        @pl.when(w0 + 2 < NWIN)
        def _():
            fire(w0 + 2, 0)
        s = process_window(w0 + 1, 1, s)
        @pl.when(w0 + 3 < NWIN)
        def _():
            fire(w0 + 3, 1)
        return s

    s = lax.fori_loop(0, (NWIN - 1) // 2, outer, jnp.int32(0))
    s = process_window(jnp.int32(NWIN - 1), 0, s)
    flush(s)


def kernel(data, indices):
    idx = indices.astype(jnp.int32)
    padded = _sc_gather_stream(data.T, idx)
    return padded[:B, :D]
